# baseline (speedup 1.0000x reference)
import functools

import jax
import jax.numpy as jnp
from jax import lax
from jax.experimental import pallas as pl
from jax.experimental.pallas import tpu as pltpu
from jax.experimental.pallas import tpu_sc as plsc

R, C = 128, 32768
LANES = 16
GROUP = 4 * LANES
CHUNK = 8192
NCHUNK = C // CHUNK


def _build_sc_kernel():
    mesh = plsc.VectorSubcoreMesh(core_axis_name="c", subcore_axis_name="s")
    info = plsc.get_sparse_core_info()
    nc, ns = info.num_cores, info.num_subcores
    nw = nc * ns
    rpw = R // nw
    T = rpw * NCHUNK

    @functools.partial(
        pl.kernel,
        mesh=mesh,
        compiler_params=pltpu.CompilerParams(needs_layout_passes=False),
        out_type=jax.ShapeDtypeStruct((R, C), jnp.float32),
        scratch_types=[
            pltpu.VMEM((CHUNK,), jnp.float32),
            pltpu.VMEM((CHUNK,), jnp.float32),
            pltpu.VMEM((CHUNK // 4,), jnp.int32),
            pltpu.VMEM((CHUNK // 4,), jnp.int32),
            pltpu.VMEM((CHUNK,), jnp.float32),
            pltpu.VMEM((CHUNK,), jnp.float32),
            pltpu.SemaphoreType.DMA,
            pltpu.SemaphoreType.DMA,
            pltpu.SemaphoreType.DMA,
            pltpu.SemaphoreType.DMA,
            pltpu.SemaphoreType.DMA,
            pltpu.SemaphoreType.DMA,
        ],
    )
    def k(x_hbm, m_hbm, out_hbm, x_v0, x_v1, m_v0, m_v1, o_v0, o_v1,
          sx0, sx1, sm0, sm1, so0, so1):
        x_v, m_v, o_v = (x_v0, x_v1), (m_v0, m_v1), (o_v0, o_v1)
        sx, sm, so = (sx0, sx1), (sm0, sm1), (so0, so1)
        wid = lax.axis_index("s") * nc + lax.axis_index("c")
        row0 = wid * rpw

        def in_slices(t):
            row = row0 + t // NCHUNK
            base = (t % NCHUNK) * CHUNK
            return row, base

        def m_off(row, base):
            return pl.multiple_of(row * (C // 4) + base // 4, CHUNK // 4)

        def start_load(t, b):
            row, base = in_slices(t)
            pltpu.async_copy(x_hbm.at[row, pl.ds(base, CHUNK)], x_v[b], sx[b])
            pltpu.async_copy(m_hbm.at[pl.ds(m_off(row, base), CHUNK // 4)], m_v[b], sm[b])

        def wait_load(t, b):
            row, base = in_slices(t)
            pltpu.make_async_copy(x_hbm.at[row, pl.ds(base, CHUNK)], x_v[b], sx[b]).wait()
            pltpu.make_async_copy(m_hbm.at[pl.ds(m_off(row, base), CHUNK // 4)], m_v[b], sm[b]).wait()

        def wait_store(t, b):
            row, base = in_slices(t)
            pltpu.make_async_copy(o_v[b], out_hbm.at[row, pl.ds(base, CHUNK)], so[b]).wait()

        start_load(0, 0)

        def pair_body(p, carry):
            for b in range(2):
                t = 2 * p + b
                wait_load(t, b)

                @pl.when(t + 1 < T)
                def _():
                    start_load(t + 1, 1 - b)

                @pl.when(t >= 2)
                def _():
                    wait_store(t, b)

                carry = jnp.where(t % NCHUNK == 0, jnp.float32(0.0), carry)
                xb, mb, ob = x_v[b], m_v[b], o_v[b]

                @plsc.parallel_loop(0, CHUNK // GROUP, unroll=2, carry=carry)
                def vec_body(g, cy):
                    off64 = g * GROUP
                    w = mb[pl.ds(g * LANES, LANES)]
                    for sub in range(4):
                        off = off64 + sub * LANES
                        bitf = ((w >> jnp.int32(8 * sub)) & jnp.int32(1)).astype(jnp.float32)
                        s = plsc.cumsum(xb[pl.ds(off, LANES)] * bitf)
                        ob[pl.ds(off, LANES)] = s + cy
                        cy = cy + s[15]
                    return cy

                carry = vec_body
                row, base = in_slices(t)
                pltpu.async_copy(o_v[b], out_hbm.at[row, pl.ds(base, CHUNK)], so[b])
            return carry

        lax.fori_loop(0, T // 2, pair_body, jnp.float32(0.0))
        wait_store(T - 2, 0)
        wait_store(T - 1, 1)

    return k


_sc_kernel = _build_sc_kernel()


@jax.jit
def kernel(x, mask):
    mt = (mask.astype(jnp.uint8)
          .reshape(R, C // GROUP, 4, LANES)
          .swapaxes(2, 3))
    mw = jax.lax.bitcast_convert_type(mt, jnp.int32).reshape(R * (C // 4))
    return _sc_kernel(x, mw)


# speedup vs baseline: 1.0116x; 1.0116x over previous
import functools

import jax
import jax.numpy as jnp
from jax import lax
from jax.experimental import pallas as pl
from jax.experimental.pallas import tpu as pltpu
from jax.experimental.pallas import tpu_sc as plsc

R, C = 128, 32768
LANES = 16
GROUP = 4 * LANES
CHUNK = 8192
NCHUNK = C // CHUNK


def _build_sc_kernel():
    mesh = plsc.VectorSubcoreMesh(core_axis_name="c", subcore_axis_name="s")
    info = plsc.get_sparse_core_info()
    nc, ns = info.num_cores, info.num_subcores
    nw = nc * ns
    rpw = R // nw
    T = rpw * NCHUNK

    @functools.partial(
        pl.kernel,
        mesh=mesh,
        compiler_params=pltpu.CompilerParams(needs_layout_passes=False),
        out_type=jax.ShapeDtypeStruct((R, C), jnp.float32),
        scratch_types=[
            pltpu.VMEM((CHUNK,), jnp.float32),
            pltpu.VMEM((CHUNK,), jnp.float32),
            pltpu.VMEM((CHUNK // 4,), jnp.int32),
            pltpu.VMEM((CHUNK // 4,), jnp.int32),
            pltpu.VMEM((CHUNK,), jnp.float32),
            pltpu.VMEM((CHUNK,), jnp.float32),
            pltpu.SemaphoreType.DMA,
            pltpu.SemaphoreType.DMA,
            pltpu.SemaphoreType.DMA,
            pltpu.SemaphoreType.DMA,
            pltpu.SemaphoreType.DMA,
            pltpu.SemaphoreType.DMA,
        ],
    )
    def k(x_hbm, m_hbm, out_hbm, x_v0, x_v1, m_v0, m_v1, o_v0, o_v1,
          sx0, sx1, sm0, sm1, so0, so1):
        x_v, m_v, o_v = (x_v0, x_v1), (m_v0, m_v1), (o_v0, o_v1)
        sx, sm, so = (sx0, sx1), (sm0, sm1), (so0, so1)
        wid = lax.axis_index("s") * nc + lax.axis_index("c")
        row0 = wid * rpw

        def in_slices(t):
            row = row0 + t // NCHUNK
            base = (t % NCHUNK) * CHUNK
            return row, base

        def m_off(row, base):
            return pl.multiple_of(row * (C // 4) + base // 4, CHUNK // 4)

        def start_load(t, b):
            row, base = in_slices(t)
            pltpu.async_copy(x_hbm.at[row, pl.ds(base, CHUNK)], x_v[b], sx[b])
            pltpu.async_copy(m_hbm.at[pl.ds(m_off(row, base), CHUNK // 4)], m_v[b], sm[b])

        def wait_load(t, b):
            row, base = in_slices(t)
            pltpu.make_async_copy(x_hbm.at[row, pl.ds(base, CHUNK)], x_v[b], sx[b]).wait()
            pltpu.make_async_copy(m_hbm.at[pl.ds(m_off(row, base), CHUNK // 4)], m_v[b], sm[b]).wait()

        def wait_store(t, b):
            row, base = in_slices(t)
            pltpu.make_async_copy(o_v[b], out_hbm.at[row, pl.ds(base, CHUNK)], so[b]).wait()

        start_load(0, 0)

        def pair_body(p, carry):
            for b in range(2):
                t = 2 * p + b
                wait_load(t, b)

                @pl.when(t + 1 < T)
                def _():
                    start_load(t + 1, 1 - b)

                @pl.when(t >= 2)
                def _():
                    wait_store(t, b)

                carry = jnp.where(t % NCHUNK == 0, jnp.float32(0.0), carry)
                xb, mb, ob = x_v[b], m_v[b], o_v[b]

                zero = jnp.zeros((LANES,), jnp.float32)

                @plsc.parallel_loop(0, CHUNK // GROUP, unroll=8, carry=carry)
                def vec_body(g, cy):
                    off64 = g * GROUP
                    w = mb[pl.ds(g * LANES, LANES)]
                    for sub in range(4):
                        off = off64 + sub * LANES
                        bit = (w & jnp.int32(1 << (8 * sub))) != 0
                        s = plsc.cumsum(jnp.where(bit, xb[pl.ds(off, LANES)], zero))
                        ob[pl.ds(off, LANES)] = s + cy
                        cy = cy + s[15]
                    return cy

                carry = vec_body
                row, base = in_slices(t)
                pltpu.async_copy(o_v[b], out_hbm.at[row, pl.ds(base, CHUNK)], so[b])
            return carry

        lax.fori_loop(0, T // 2, pair_body, jnp.float32(0.0))
        wait_store(T - 2, 0)
        wait_store(T - 1, 1)

    return k


_sc_kernel = _build_sc_kernel()


@jax.jit
def kernel(x, mask):
    mt = (mask.astype(jnp.uint8)
          .reshape(R, C // GROUP, 4, LANES)
          .swapaxes(2, 3))
    mw = jax.lax.bitcast_convert_type(mt, jnp.int32).reshape(R * (C // 4))
    return _sc_kernel(x, mw)


# speedup vs baseline: 3.0057x; 2.9713x over previous
import functools

import jax
import jax.numpy as jnp
from jax import lax
from jax.experimental import pallas as pl
from jax.experimental.pallas import tpu as pltpu
from jax.experimental.pallas import tpu_sc as plsc

R, C = 128, 32768
LANES = 16
CHUNK = C // 4
NCHUNK = C // CHUNK


def _build_sc_kernel():
    mesh = plsc.VectorSubcoreMesh(core_axis_name="c", subcore_axis_name="s")
    info = plsc.get_sparse_core_info()
    nc, ns = info.num_cores, info.num_subcores
    nw = nc * ns
    rpw = R // nw
    T = rpw * NCHUNK

    @functools.partial(
        pl.kernel,
        mesh=mesh,
        compiler_params=pltpu.CompilerParams(needs_layout_passes=False),
        out_type=jax.ShapeDtypeStruct((R, C), jnp.float32),
        scratch_types=[
            pltpu.VMEM((CHUNK,), jnp.float32),
            pltpu.VMEM((CHUNK,), jnp.float32),
            pltpu.VMEM((CHUNK,), jnp.int32),
            pltpu.VMEM((CHUNK,), jnp.int32),
            pltpu.VMEM((CHUNK,), jnp.float32),
            pltpu.VMEM((CHUNK,), jnp.float32),
            pltpu.SemaphoreType.DMA,
            pltpu.SemaphoreType.DMA,
            pltpu.SemaphoreType.DMA,
            pltpu.SemaphoreType.DMA,
            pltpu.SemaphoreType.DMA,
            pltpu.SemaphoreType.DMA,
        ],
    )
    def k(x_hbm, m_hbm, out_hbm, x_v0, x_v1, m_v0, m_v1, o_v0, o_v1,
          sx0, sx1, sm0, sm1, so0, so1):
        x_v, m_v, o_v = (x_v0, x_v1), (m_v0, m_v1), (o_v0, o_v1)
        sx, sm, so = (sx0, sx1), (sm0, sm1), (so0, so1)
        wid = lax.axis_index("s") * nc + lax.axis_index("c")
        row0 = wid * rpw

        def x_slc(t):
            return x_hbm.at[row0 + t // NCHUNK, pl.ds((t % NCHUNK) * CHUNK, CHUNK)]

        def o_slc(t):
            return out_hbm.at[row0 + t // NCHUNK, pl.ds((t % NCHUNK) * CHUNK, CHUNK)]

        def m_slc(r):
            return m_hbm.at[pl.ds(pl.multiple_of((row0 + r) * CHUNK, CHUNK), CHUNK)]

        pltpu.async_copy(x_slc(0), x_v[0], sx[0])
        pltpu.async_copy(m_slc(0), m_v[0], sm[0])

        zero = jnp.zeros((LANES,), jnp.float32)
        carry = jnp.float32(0.0)

        for t in range(T):
            b = t % 2
            q = t % NCHUNK
            r = t // NCHUNK
            mbuf = r % 2

            pltpu.make_async_copy(x_slc(t), x_v[b], sx[b]).wait()
            if t + 1 < T:
                pltpu.async_copy(x_slc(t + 1), x_v[1 - b], sx[1 - b])
            if q == 0:
                if r + 1 < rpw:
                    pltpu.async_copy(m_slc(r + 1), m_v[1 - mbuf], sm[1 - mbuf])
                pltpu.make_async_copy(m_slc(r), m_v[mbuf], sm[mbuf]).wait()
                carry = jnp.float32(0.0)
            if t >= 2:
                pltpu.make_async_copy(o_v[b], o_slc(t - 2), so[b]).wait()

            xb, mb, ob = x_v[b], m_v[mbuf], o_v[b]
            bitmask = jnp.int32(1 << (8 * q))

            @plsc.parallel_loop(0, CHUNK // LANES, unroll=8, carry=carry)
            def vec_body(g, cy):
                off = g * LANES
                bit = (mb[pl.ds(off, LANES)] & bitmask) != 0
                s = plsc.cumsum(jnp.where(bit, xb[pl.ds(off, LANES)], zero))
                ob[pl.ds(off, LANES)] = s + cy
                return cy + s[15]

            carry = vec_body
            pltpu.async_copy(o_v[b], o_slc(t), so[b])

        pltpu.make_async_copy(o_v[0], o_slc(T - 2), so[0]).wait()
        pltpu.make_async_copy(o_v[1], o_slc(T - 1), so[1]).wait()

    return k


_sc_kernel = _build_sc_kernel()


@jax.jit
def kernel(x, mask):
    mi = mask.astype(jnp.int32)
    mw = (mi[:, :CHUNK]
          | (mi[:, CHUNK:2 * CHUNK] << 8)
          | (mi[:, 2 * CHUNK:3 * CHUNK] << 16)
          | (mi[:, 3 * CHUNK:] << 24))
    return _sc_kernel(x, mw.reshape(R * CHUNK))


# speedup vs baseline: 3.8150x; 1.2692x over previous
import functools

import jax
import jax.numpy as jnp
from jax import lax
from jax.experimental import pallas as pl
from jax.experimental.pallas import tpu as pltpu
from jax.experimental.pallas import tpu_sc as plsc

R, C = 128, 32768
LANES = 16
CHUNK = C // 4
NCHUNK = C // CHUNK


def _build_sc_kernel():
    mesh = plsc.VectorSubcoreMesh(core_axis_name="c", subcore_axis_name="s")
    info = plsc.get_sparse_core_info()
    nc, ns = info.num_cores, info.num_subcores
    nw = nc * ns
    rpw = R // nw
    T = rpw * NCHUNK

    @functools.partial(
        pl.kernel,
        mesh=mesh,
        compiler_params=pltpu.CompilerParams(needs_layout_passes=False),
        out_type=jax.ShapeDtypeStruct((R, C), jnp.float32),
        scratch_types=[
            pltpu.VMEM((CHUNK,), jnp.float32),
            pltpu.VMEM((CHUNK,), jnp.float32),
            pltpu.VMEM((CHUNK,), jnp.int32),
            pltpu.VMEM((CHUNK,), jnp.int32),
            pltpu.VMEM((CHUNK,), jnp.float32),
            pltpu.VMEM((CHUNK,), jnp.float32),
            pltpu.SemaphoreType.DMA,
            pltpu.SemaphoreType.DMA,
            pltpu.SemaphoreType.DMA,
            pltpu.SemaphoreType.DMA,
            pltpu.SemaphoreType.DMA,
            pltpu.SemaphoreType.DMA,
        ],
    )
    def k(x_hbm, m_hbm, out_hbm, x_v0, x_v1, m_v0, m_v1, o_v0, o_v1,
          sx0, sx1, sm0, sm1, so0, so1):
        x_v, m_v, o_v = (x_v0, x_v1), (m_v0, m_v1), (o_v0, o_v1)
        sx, sm, so = (sx0, sx1), (sm0, sm1), (so0, so1)
        wid = lax.axis_index("s") * nc + lax.axis_index("c")
        row0 = wid * rpw

        def x_slc(t):
            return x_hbm.at[row0 + t // NCHUNK, pl.ds((t % NCHUNK) * CHUNK, CHUNK)]

        def o_slc(t):
            return out_hbm.at[row0 + t // NCHUNK, pl.ds((t % NCHUNK) * CHUNK, CHUNK)]

        def m_slc(r):
            return m_hbm.at[pl.ds(pl.multiple_of((row0 + r) * CHUNK, CHUNK), CHUNK)]

        pltpu.async_copy(x_slc(0), x_v[0], sx[0])
        pltpu.async_copy(m_slc(0), m_v[0], sm[0])

        zero = jnp.zeros((LANES,), jnp.float32)
        carry = jnp.float32(0.0)

        for t in range(T):
            b = t % 2
            q = t % NCHUNK
            r = t // NCHUNK
            mbuf = r % 2

            pltpu.make_async_copy(x_slc(t), x_v[b], sx[b]).wait()
            if t + 1 < T:
                pltpu.async_copy(x_slc(t + 1), x_v[1 - b], sx[1 - b])
            if q == 0:
                if r + 1 < rpw:
                    pltpu.async_copy(m_slc(r + 1), m_v[1 - mbuf], sm[1 - mbuf])
                pltpu.make_async_copy(m_slc(r), m_v[mbuf], sm[mbuf]).wait()
                carry = jnp.float32(0.0)
            if t >= 2:
                pltpu.make_async_copy(o_v[b], o_slc(t - 2), so[b]).wait()

            xb, mb, ob = x_v[b], m_v[mbuf], o_v[b]
            bitmask = jnp.int32(1 << (8 * q))

            @plsc.parallel_loop(0, CHUNK // LANES, unroll=8, carry=carry)
            def vec_body(g, cy):
                off = g * LANES
                bit = (mb[pl.ds(off, LANES)] & bitmask) != 0
                s = plsc.cumsum(jnp.where(bit, xb[pl.ds(off, LANES)], zero))
                ob[pl.ds(off, LANES)] = s + cy
                return cy + s[15]

            carry = vec_body
            pltpu.async_copy(o_v[b], o_slc(t), so[b])

        pltpu.make_async_copy(o_v[0], o_slc(T - 2), so[0]).wait()
        pltpu.make_async_copy(o_v[1], o_slc(T - 1), so[1]).wait()

    return k


_sc_kernel = _build_sc_kernel()


@jax.jit
def kernel(x, mask):
    mw = (mask[:, :CHUNK].astype(jnp.int32)
          | (mask[:, CHUNK:2 * CHUNK].astype(jnp.int32) << 8)
          | (mask[:, 2 * CHUNK:3 * CHUNK].astype(jnp.int32) << 16)
          | (mask[:, 3 * CHUNK:].astype(jnp.int32) << 24))
    return _sc_kernel(x, mw.reshape(R * CHUNK))
